# Initial kernel scaffold; baseline (speedup 1.0000x reference)
import jax
import jax.numpy as jnp
from jax.experimental import pallas as pl


def kernel(x, edge_index, W1, b1, W2, b2):
    raise NotImplementedError("write your pallas kernel here")


# speedup vs baseline: 12.5219x; 12.5219x over previous
import functools

import jax
import jax.numpy as jnp
from jax import lax
from jax.experimental import pallas as pl
from jax.experimental.pallas import tpu as pltpu
from jax.experimental.pallas import tpu_sc as plsc

N = 10000
E = 320000
D = 128

NC = 2
NS = 16
NW = NC * NS
EPW = E // NW
CH = 80
NCHUNK = EPW // CH
RPT = 624
TAIL = N - NS * RPT
DEGW = 8

_MESH = plsc.VectorSubcoreMesh(core_axis_name="c", subcore_axis_name="s")


@functools.partial(
    pl.kernel,
    out_type=jax.ShapeDtypeStruct((NC, N, DEGW), jnp.float32),
    mesh=_MESH,
    scratch_types=[
        pltpu.VMEM((CH,), jnp.int32),
        pltpu.VMEM((CH, DEGW), jnp.float32),
        pltpu.VMEM_SHARED((N, DEGW), jnp.float32),
    ],
)
def _sc_degree(cols, zeros8, ones8, deg_out, cidx, ones_v, deg_sp):
    c = lax.axis_index("c")
    s = lax.axis_index("s")
    wid = c * NS + s
    base = wid * EPW
    pltpu.sync_copy(zeros8.at[pl.ds(s * RPT, RPT)], deg_sp.at[pl.ds(s * RPT, RPT)])
    @pl.when(s == NS - 1)
    def _():
        pltpu.sync_copy(zeros8.at[pl.ds(NS * RPT, TAIL)],
                        deg_sp.at[pl.ds(NS * RPT, TAIL)])
    pltpu.sync_copy(ones8, ones_v)
    plsc.subcore_barrier()

    def body(j, carry):
        pltpu.sync_copy(cols.at[pl.ds(base + j * CH, CH)], cidx)
        pltpu.sync_copy(ones_v, deg_sp.at[cidx], add=True)
        return carry

    lax.fori_loop(0, NCHUNK, body, 0)
    plsc.subcore_barrier()
    pltpu.sync_copy(deg_sp.at[pl.ds(s * RPT, RPT)],
                    deg_out.at[c, pl.ds(s * RPT, RPT)])
    @pl.when(s == NS - 1)
    def _():
        pltpu.sync_copy(deg_sp.at[pl.ds(NS * RPT, TAIL)],
                        deg_out.at[c, pl.ds(NS * RPT, TAIL)])


@functools.partial(
    pl.kernel,
    out_type=jax.ShapeDtypeStruct((NC, N, D), jnp.float32),
    mesh=_MESH,
    scratch_types=[
        pltpu.VMEM((CH,), jnp.int32),
        pltpu.VMEM((CH,), jnp.int32),
        pltpu.VMEM((CH, D), jnp.float32),
        pltpu.VMEM_SHARED((N, D), jnp.float32),
        pltpu.SemaphoreType.DMA,
    ],
)
def _sc_scatter(g, rows, cols, zerosd, s_out, ridx, cidx, buf, s_sp, sem):
    c = lax.axis_index("c")
    s = lax.axis_index("s")
    wid = c * NS + s
    base = wid * EPW
    pltpu.sync_copy(zerosd.at[pl.ds(s * RPT, RPT)], s_sp.at[pl.ds(s * RPT, RPT)])
    @pl.when(s == NS - 1)
    def _():
        pltpu.sync_copy(zerosd.at[pl.ds(NS * RPT, TAIL)],
                        s_sp.at[pl.ds(NS * RPT, TAIL)])
    plsc.subcore_barrier()

    def body(j, carry):
        pltpu.sync_copy(rows.at[pl.ds(base + j * CH, CH)], ridx)
        pltpu.sync_copy(cols.at[pl.ds(base + j * CH, CH)], cidx)
        pltpu.async_copy(g.at[ridx], buf, sem).wait()
        pltpu.sync_copy(buf, s_sp.at[cidx], add=True)
        return carry

    lax.fori_loop(0, NCHUNK, body, 0)
    plsc.subcore_barrier()
    pltpu.sync_copy(s_sp.at[pl.ds(s * RPT, RPT)],
                    s_out.at[c, pl.ds(s * RPT, RPT)])
    @pl.when(s == NS - 1)
    def _():
        pltpu.sync_copy(s_sp.at[pl.ds(NS * RPT, TAIL)],
                        s_out.at[c, pl.ds(NS * RPT, TAIL)])


_BLK = 1000
_GRID = N // _BLK

_row_spec = pl.BlockSpec((_BLK, D), lambda i: (i, 0))
_deg_spec = pl.BlockSpec((_BLK, DEGW), lambda i: (i, 0))
_w_spec = pl.BlockSpec((D, D), lambda i: (0, 0))
_b_spec = pl.BlockSpec((1, D), lambda i: (0, 0))


def _dinv(dega_ref, degb_ref):
    d = dega_ref[:, 0:1] + degb_ref[:, 0:1] + 1.0
    return lax.rsqrt(d)


def _tc_prep_body(dega_ref, degb_ref, x_ref, w_ref, g_ref):
    dinv = _dinv(dega_ref, degb_ref)
    h = jnp.dot(x_ref[...], w_ref[...], preferred_element_type=jnp.float32)
    g_ref[...] = dinv * h


_tc_prep = pl.pallas_call(
    _tc_prep_body,
    grid=(_GRID,),
    in_specs=[_deg_spec, _deg_spec, _row_spec, _w_spec],
    out_specs=_row_spec,
    out_shape=jax.ShapeDtypeStruct((N, D), jnp.float32),
)


def _tc_mid_body(sa_ref, sb_ref, g1_ref, dega_ref, degb_ref, b_ref, w_ref, g2_ref):
    dinv = _dinv(dega_ref, degb_ref)
    s = sa_ref[...] + sb_ref[...] + g1_ref[...]
    z = jnp.maximum(dinv * s + b_ref[...], 0.0)
    h = jnp.dot(z, w_ref[...], preferred_element_type=jnp.float32)
    g2_ref[...] = dinv * h


_tc_mid = pl.pallas_call(
    _tc_mid_body,
    grid=(_GRID,),
    in_specs=[_row_spec, _row_spec, _row_spec, _deg_spec, _deg_spec, _b_spec, _w_spec],
    out_specs=_row_spec,
    out_shape=jax.ShapeDtypeStruct((N, D), jnp.float32),
)


def _tc_final_body(sa_ref, sb_ref, g2_ref, dega_ref, degb_ref, b_ref, o_ref):
    dinv = _dinv(dega_ref, degb_ref)
    s = sa_ref[...] + sb_ref[...] + g2_ref[...]
    o_ref[...] = dinv * s + b_ref[...]


_tc_final = pl.pallas_call(
    _tc_final_body,
    grid=(_GRID,),
    in_specs=[_row_spec, _row_spec, _row_spec, _deg_spec, _deg_spec, _b_spec],
    out_specs=_row_spec,
    out_shape=jax.ShapeDtypeStruct((N, D), jnp.float32),
)


def kernel(x, edge_index, W1, b1, W2, b2):
    rows = edge_index[0]
    cols = edge_index[1]
    zeros8 = jnp.zeros((N, DEGW), jnp.float32)
    ones8 = jnp.ones((CH, DEGW), jnp.float32)
    zerosd = jnp.zeros((N, D), jnp.float32)
    b1r = b1.reshape(1, D)
    b2r = b2.reshape(1, D)

    deg_p = _sc_degree(cols, zeros8, ones8)
    dega, degb = deg_p[0], deg_p[1]

    g1 = _tc_prep(dega, degb, x, W1)
    s1 = _sc_scatter(g1, rows, cols, zerosd)
    g2 = _tc_mid(s1[0], s1[1], g1, dega, degb, b1r, W2)
    s2 = _sc_scatter(g2, rows, cols, zerosd)
    return _tc_final(s2[0], s2[1], g2, dega, degb, b2r)


# speedup vs baseline: 21.7692x; 1.7385x over previous
import functools

import jax
import jax.numpy as jnp
from jax import lax
from jax.experimental import pallas as pl
from jax.experimental.pallas import tpu as pltpu
from jax.experimental.pallas import tpu_sc as plsc

N = 10000
E = 320000
D = 128

NC = 2
NS = 16
NW = NC * NS
EPW = E // NW
CH = 80
NCHUNK = EPW // CH
RPT = 624
TAIL = N - NS * RPT
DEGW = 8

_MESH = plsc.VectorSubcoreMesh(core_axis_name="c", subcore_axis_name="s")


@functools.partial(
    pl.kernel,
    out_type=jax.ShapeDtypeStruct((NC, N, DEGW), jnp.float32),
    mesh=_MESH,
    scratch_types=[
        pltpu.VMEM((CH,), jnp.int32),
        pltpu.VMEM((CH,), jnp.int32),
        pltpu.VMEM((CH, DEGW), jnp.float32),
        pltpu.SemaphoreType.DMA,
        pltpu.SemaphoreType.DMA,
        pltpu.SemaphoreType.DMA,
        pltpu.SemaphoreType.DMA,
        pltpu.VMEM_SHARED((N, DEGW), jnp.float32),
    ],
)
def _sc_degree(cols, zeros8, ones8, deg_out,
               cidx0, cidx1, ones_v, isem0, isem1, ssem0, ssem1, deg_sp):
    c = lax.axis_index("c")
    s = lax.axis_index("s")
    wid = c * NS + s
    base = wid * EPW
    pltpu.sync_copy(zeros8.at[pl.ds(s * RPT, RPT)], deg_sp.at[pl.ds(s * RPT, RPT)])
    @pl.when(s == NS - 1)
    def _():
        pltpu.sync_copy(zeros8.at[pl.ds(NS * RPT, TAIL)],
                        deg_sp.at[pl.ds(NS * RPT, TAIL)])
    pltpu.sync_copy(ones8, ones_v)
    plsc.subcore_barrier()

    cidx = (cidx0, cidx1)
    isem = (isem0, isem1)
    ssem = (ssem0, ssem1)

    pltpu.sync_copy(cols.at[pl.ds(base, CH)], cidx0)
    pltpu.async_copy(ones_v, deg_sp.at[cidx0], ssem0, add=True)
    pltpu.async_copy(cols.at[pl.ds(base + CH, CH)], cidx1, isem1)

    def step(ch, b, load_next):
        pltpu.make_async_copy(cols.at[pl.ds(base + ch * CH, CH)],
                              cidx[b], isem[b]).wait()
        pltpu.async_copy(ones_v, deg_sp.at[cidx[b]], ssem[b], add=True)
        pltpu.make_async_copy(ones_v, deg_sp.at[cidx[1 - b]],
                              ssem[1 - b]).wait()
        @pl.when(load_next)
        def _():
            pltpu.async_copy(cols.at[pl.ds(base + (ch + 1) * CH, CH)],
                             cidx[1 - b], isem[1 - b])

    def body(i, carry):
        ch = 2 * i + 1
        step(ch, 1, ch + 1 < NCHUNK)
        step(ch + 1, 0, ch + 2 < NCHUNK)
        return carry

    lax.fori_loop(0, (NCHUNK - 1) // 2, body, 0)
    last = NCHUNK - 1
    pltpu.make_async_copy(ones_v, deg_sp.at[cidx[last % 2]],
                          ssem[last % 2]).wait()
    plsc.subcore_barrier()
    pltpu.sync_copy(deg_sp.at[pl.ds(s * RPT, RPT)],
                    deg_out.at[c, pl.ds(s * RPT, RPT)])
    @pl.when(s == NS - 1)
    def _():
        pltpu.sync_copy(deg_sp.at[pl.ds(NS * RPT, TAIL)],
                        deg_out.at[c, pl.ds(NS * RPT, TAIL)])


@functools.partial(
    pl.kernel,
    out_type=jax.ShapeDtypeStruct((NC, N, D), jnp.float32),
    mesh=_MESH,
    scratch_types=[
        pltpu.VMEM((CH,), jnp.int32),
        pltpu.VMEM((CH,), jnp.int32),
        pltpu.VMEM((CH,), jnp.int32),
        pltpu.VMEM((CH,), jnp.int32),
        pltpu.VMEM((CH, D), jnp.float32),
        pltpu.VMEM((CH, D), jnp.float32),
        pltpu.SemaphoreType.DMA,
        pltpu.SemaphoreType.DMA,
        pltpu.SemaphoreType.DMA,
        pltpu.SemaphoreType.DMA,
        pltpu.VMEM_SHARED((N, D), jnp.float32),
    ],
)
def _sc_scatter(g, rows, cols, zerosd, s_out,
                ridx0, ridx1, cidx0, cidx1, buf0, buf1,
                gsem0, gsem1, isem0, isem1, s_sp):
    c = lax.axis_index("c")
    s = lax.axis_index("s")
    wid = c * NS + s
    base = wid * EPW
    pltpu.sync_copy(zerosd.at[pl.ds(s * RPT, RPT)], s_sp.at[pl.ds(s * RPT, RPT)])
    @pl.when(s == NS - 1)
    def _():
        pltpu.sync_copy(zerosd.at[pl.ds(NS * RPT, TAIL)],
                        s_sp.at[pl.ds(NS * RPT, TAIL)])
    plsc.subcore_barrier()

    ridx = (ridx0, ridx1)
    cidx = (cidx0, cidx1)
    bufs = (buf0, buf1)
    gsem = (gsem0, gsem1)
    isem = (isem0, isem1)

    pltpu.sync_copy(rows.at[pl.ds(base, CH)], ridx0)
    pltpu.sync_copy(cols.at[pl.ds(base, CH)], cidx0)
    pltpu.async_copy(g.at[ridx0], buf0, gsem0)
    pltpu.async_copy(rows.at[pl.ds(base + CH, CH)], ridx1, isem1)
    pltpu.async_copy(cols.at[pl.ds(base + CH, CH)], cidx1, isem1)

    def step(ch, b, load_next):
        pltpu.make_async_copy(rows.at[pl.ds(base + ch * CH, CH)],
                              ridx[b], isem[b]).wait()
        pltpu.make_async_copy(cols.at[pl.ds(base + ch * CH, CH)],
                              cidx[b], isem[b]).wait()
        pltpu.async_copy(g.at[ridx[b]], bufs[b], gsem[b])
        pltpu.make_async_copy(g.at[ridx[1 - b]], bufs[1 - b],
                              gsem[1 - b]).wait()
        pltpu.sync_copy(bufs[1 - b], s_sp.at[cidx[1 - b]], add=True)
        @pl.when(load_next)
        def _():
            pltpu.async_copy(rows.at[pl.ds(base + (ch + 1) * CH, CH)],
                             ridx[1 - b], isem[1 - b])
            pltpu.async_copy(cols.at[pl.ds(base + (ch + 1) * CH, CH)],
                             cidx[1 - b], isem[1 - b])

    def body(i, carry):
        ch = 2 * i + 1
        step(ch, 1, ch + 1 < NCHUNK)
        step(ch + 1, 0, ch + 2 < NCHUNK)
        return carry

    lax.fori_loop(0, (NCHUNK - 1) // 2, body, 0)
    last = NCHUNK - 1
    pltpu.make_async_copy(g.at[ridx[last % 2]], bufs[last % 2],
                          gsem[last % 2]).wait()
    pltpu.sync_copy(bufs[last % 2], s_sp.at[cidx[last % 2]], add=True)
    plsc.subcore_barrier()
    pltpu.sync_copy(s_sp.at[pl.ds(s * RPT, RPT)],
                    s_out.at[c, pl.ds(s * RPT, RPT)])
    @pl.when(s == NS - 1)
    def _():
        pltpu.sync_copy(s_sp.at[pl.ds(NS * RPT, TAIL)],
                        s_out.at[c, pl.ds(NS * RPT, TAIL)])


_BLK = 1000
_GRID = N // _BLK

_row_spec = pl.BlockSpec((_BLK, D), lambda i: (i, 0))
_deg_spec = pl.BlockSpec((_BLK, DEGW), lambda i: (i, 0))
_w_spec = pl.BlockSpec((D, D), lambda i: (0, 0))
_b_spec = pl.BlockSpec((1, D), lambda i: (0, 0))


def _dinv(dega_ref, degb_ref):
    d = dega_ref[:, 0:1] + degb_ref[:, 0:1] + 1.0
    return lax.rsqrt(d)


def _tc_prep_body(dega_ref, degb_ref, x_ref, w_ref, g_ref):
    dinv = _dinv(dega_ref, degb_ref)
    h = jnp.dot(x_ref[...], w_ref[...], preferred_element_type=jnp.float32)
    g_ref[...] = dinv * h


_tc_prep = pl.pallas_call(
    _tc_prep_body,
    grid=(_GRID,),
    in_specs=[_deg_spec, _deg_spec, _row_spec, _w_spec],
    out_specs=_row_spec,
    out_shape=jax.ShapeDtypeStruct((N, D), jnp.float32),
)


def _tc_mid_body(sa_ref, sb_ref, g1_ref, dega_ref, degb_ref, b_ref, w_ref, g2_ref):
    dinv = _dinv(dega_ref, degb_ref)
    s = sa_ref[...] + sb_ref[...] + g1_ref[...]
    z = jnp.maximum(dinv * s + b_ref[...], 0.0)
    h = jnp.dot(z, w_ref[...], preferred_element_type=jnp.float32)
    g2_ref[...] = dinv * h


_tc_mid = pl.pallas_call(
    _tc_mid_body,
    grid=(_GRID,),
    in_specs=[_row_spec, _row_spec, _row_spec, _deg_spec, _deg_spec, _b_spec, _w_spec],
    out_specs=_row_spec,
    out_shape=jax.ShapeDtypeStruct((N, D), jnp.float32),
)


def _tc_final_body(sa_ref, sb_ref, g2_ref, dega_ref, degb_ref, b_ref, o_ref):
    dinv = _dinv(dega_ref, degb_ref)
    s = sa_ref[...] + sb_ref[...] + g2_ref[...]
    o_ref[...] = dinv * s + b_ref[...]


_tc_final = pl.pallas_call(
    _tc_final_body,
    grid=(_GRID,),
    in_specs=[_row_spec, _row_spec, _row_spec, _deg_spec, _deg_spec, _b_spec],
    out_specs=_row_spec,
    out_shape=jax.ShapeDtypeStruct((N, D), jnp.float32),
)


def kernel(x, edge_index, W1, b1, W2, b2):
    rows = edge_index[0]
    cols = edge_index[1]
    zeros8 = jnp.zeros((N, DEGW), jnp.float32)
    ones8 = jnp.ones((CH, DEGW), jnp.float32)
    zerosd = jnp.zeros((N, D), jnp.float32)
    b1r = b1.reshape(1, D)
    b2r = b2.reshape(1, D)

    deg_p = _sc_degree(cols, zeros8, ones8)
    dega, degb = deg_p[0], deg_p[1]

    g1 = _tc_prep(dega, degb, x, W1)
    s1 = _sc_scatter(g1, rows, cols, zerosd)
    g2 = _tc_mid(s1[0], s1[1], g1, dega, degb, b1r, W2)
    s2 = _sc_scatter(g2, rows, cols, zerosd)
    return _tc_final(s2[0], s2[1], g2, dega, degb, b2r)


# speedup vs baseline: 21.8558x; 1.0040x over previous
import functools

import jax
import jax.numpy as jnp
from jax import lax
from jax.experimental import pallas as pl
from jax.experimental.pallas import tpu as pltpu
from jax.experimental.pallas import tpu_sc as plsc

N = 10000
E = 320000
D = 128

NC = 2
NS = 16
NW = NC * NS
EPW = E // NW
CH = 80
NCHUNK = EPW // CH
RPT = 624
TAIL = N - NS * RPT
DEGW = 16

_MESH = plsc.VectorSubcoreMesh(core_axis_name="c", subcore_axis_name="s")


@functools.partial(
    pl.kernel,
    out_type=jax.ShapeDtypeStruct((NC, N, D), jnp.float32),
    mesh=_MESH,
    scratch_types=[
        pltpu.VMEM((CH,), jnp.int32),
        pltpu.VMEM((CH,), jnp.int32),
        pltpu.VMEM((CH, D), jnp.float32),
        pltpu.SemaphoreType.DMA,
        pltpu.SemaphoreType.DMA,
        pltpu.VMEM_SHARED((N, D), jnp.float32),
    ],
)
def _sc_degree(cols, zerosd, onesd, deg_out,
               cidx0, cidx1, ones_v, isem0, isem1, deg_sp):
    c = lax.axis_index("c")
    s = lax.axis_index("s")
    wid = c * NS + s
    base = wid * EPW
    pltpu.sync_copy(zerosd.at[pl.ds(s * RPT, RPT)], deg_sp.at[pl.ds(s * RPT, RPT)])
    @pl.when(s == NS - 1)
    def _():
        pltpu.sync_copy(zerosd.at[pl.ds(NS * RPT, TAIL)],
                        deg_sp.at[pl.ds(NS * RPT, TAIL)])
    pltpu.sync_copy(onesd, ones_v)
    cidx = (cidx0, cidx1)
    isem = (isem0, isem1)
    pltpu.sync_copy(cols.at[pl.ds(base, CH)], cidx0)
    pltpu.async_copy(cols.at[pl.ds(base + CH, CH)], cidx1, isem1)
    plsc.subcore_barrier()
    pltpu.sync_copy(ones_v, deg_sp.at[cidx0], add=True)

    def step(ch, b, load_next):
        pltpu.make_async_copy(cols.at[pl.ds(base + ch * CH, CH)],
                              cidx[b], isem[b]).wait()
        @pl.when(load_next)
        def _():
            pltpu.async_copy(cols.at[pl.ds(base + (ch + 1) * CH, CH)],
                             cidx[1 - b], isem[1 - b])
        pltpu.sync_copy(ones_v, deg_sp.at[cidx[b]], add=True)

    def body(i, carry):
        ch = 2 * i + 1
        step(ch, 1, ch + 1 < NCHUNK)
        step(ch + 1, 0, ch + 2 < NCHUNK)
        return carry

    lax.fori_loop(0, (NCHUNK - 1) // 2, body, 0)
    plsc.subcore_barrier()
    pltpu.sync_copy(deg_sp.at[pl.ds(s * RPT, RPT)],
                    deg_out.at[c, pl.ds(s * RPT, RPT)])
    @pl.when(s == NS - 1)
    def _():
        pltpu.sync_copy(deg_sp.at[pl.ds(NS * RPT, TAIL)],
                        deg_out.at[c, pl.ds(NS * RPT, TAIL)])


@functools.partial(
    pl.kernel,
    out_type=jax.ShapeDtypeStruct((NC, N, D), jnp.float32),
    mesh=_MESH,
    scratch_types=[
        pltpu.VMEM((CH,), jnp.int32),
        pltpu.VMEM((CH,), jnp.int32),
        pltpu.VMEM((CH,), jnp.int32),
        pltpu.VMEM((CH,), jnp.int32),
        pltpu.VMEM((CH,), jnp.int32),
        pltpu.VMEM((CH,), jnp.int32),
        pltpu.VMEM((CH,), jnp.int32),
        pltpu.VMEM((CH, D), jnp.float32),
        pltpu.VMEM((CH, D), jnp.float32),
        pltpu.VMEM((CH, D), jnp.float32),
        pltpu.SemaphoreType.DMA,
        pltpu.SemaphoreType.DMA,
        pltpu.SemaphoreType.DMA,
        pltpu.SemaphoreType.DMA,
        pltpu.SemaphoreType.DMA,
        pltpu.SemaphoreType.DMA,
        pltpu.SemaphoreType.DMA,
        pltpu.SemaphoreType.DMA,
        pltpu.SemaphoreType.DMA,
        pltpu.SemaphoreType.DMA,
        pltpu.SemaphoreType.DMA,
        pltpu.SemaphoreType.DMA,
        pltpu.SemaphoreType.DMA,
        pltpu.SemaphoreType.DMA,
        pltpu.VMEM_SHARED((N, D), jnp.float32),
    ],
)
def _sc_scatter(g, rows, cols, zerosd, s_out,
                ridx0, ridx1, ridx2, cidx0, cidx1, cidx2, cidx3,
                buf0, buf1, buf2,
                rsem0, rsem1, rsem2, isem0, isem1, isem2, isem3,
                gsem0, gsem1, gsem2,
                ssem0, ssem1, ssem2, ssem3,
                s_sp):
    c = lax.axis_index("c")
    s = lax.axis_index("s")
    wid = c * NS + s
    base = wid * EPW
    pltpu.sync_copy(zerosd.at[pl.ds(s * RPT, RPT)], s_sp.at[pl.ds(s * RPT, RPT)])
    @pl.when(s == NS - 1)
    def _():
        pltpu.sync_copy(zerosd.at[pl.ds(NS * RPT, TAIL)],
                        s_sp.at[pl.ds(NS * RPT, TAIL)])

    ridx = (ridx0, ridx1, ridx2)
    cidx = (cidx0, cidx1, cidx2, cidx3)
    bufs = (buf0, buf1, buf2)
    rsem = (rsem0, rsem1, rsem2)
    isem = (isem0, isem1, isem2, isem3)
    gsem = (gsem0, gsem1, gsem2)
    ssem = (ssem0, ssem1, ssem2, ssem3)

    def rload(ch, k):
        pltpu.async_copy(rows.at[pl.ds(base + ch * CH, CH)], ridx[k], rsem[k])

    def rwait(k):
        pltpu.make_async_copy(rows.at[pl.ds(base, CH)], ridx[k], rsem[k]).wait()

    def cload(ch, k):
        pltpu.async_copy(cols.at[pl.ds(base + ch * CH, CH)], cidx[k], isem[k])

    def iwait(k):
        pltpu.make_async_copy(cols.at[pl.ds(base, CH)], cidx[k], isem[k]).wait()

    def gath(k):
        pltpu.async_copy(g.at[ridx[k]], bufs[k], gsem[k])

    def gwait(k):
        pltpu.make_async_copy(g.at[ridx[k]], bufs[k], gsem[k]).wait()

    def scat(bk, ck):
        pltpu.sync_copy(bufs[bk], s_sp.at[cidx[ck]], add=True)

    def swait(bk, ck):
        pass

    pltpu.sync_copy(rows.at[pl.ds(base, CH)], ridx0)
    pltpu.sync_copy(cols.at[pl.ds(base, CH)], cidx0)
    rload(1, 1)
    rload(2, 2)
    cload(1, 1)
    plsc.subcore_barrier()

    gath(0)
    iwait(1); rwait(1); gath(1)
    gwait(0); scat(0, 0)
    cload(2, 2); rload(3, 0)
    iwait(2); rwait(2); gath(2)
    gwait(1); scat(1, 1)
    cload(3, 3); rload(4, 1)
    for ch in (3, 4):
        iwait(ch % 4); rwait(ch % 3); swait((ch - 3) % 3, (ch - 3) % 4)
        gath(ch % 3)
        gwait((ch - 1) % 3)
        scat((ch - 1) % 3, (ch - 1) % 4)
        cload(ch + 1, (ch + 1) % 4)
        rload(ch + 2, (ch + 2) % 3)

    def body(i, carry):
        ch0 = 5 + 12 * i
        for k in range(12):
            ch = ch0 + k
            c4 = (5 + k) % 4
            c3 = (5 + k) % 3
            iwait(c4)
            rwait(c3)
            swait(c3, (c4 + 1) % 4)
            gath(c3)
            gwait((c3 + 2) % 3)
            scat((c3 + 2) % 3, (c4 + 3) % 4)
            @pl.when(ch + 1 < NCHUNK)
            def _():
                cload(ch + 1, (c4 + 1) % 4)
            @pl.when(ch + 2 < NCHUNK)
            def _():
                rload(ch + 2, (c3 + 2) % 3)
        return carry

    lax.fori_loop(0, (NCHUNK - 5) // 12, body, 0)
    last = NCHUNK - 1
    gwait(last % 3)
    scat(last % 3, last % 4)
    for m in (last - 2, last - 1, last):
        swait(m % 3, m % 4)

    plsc.subcore_barrier()
    pltpu.sync_copy(s_sp.at[pl.ds(s * RPT, RPT)],
                    s_out.at[c, pl.ds(s * RPT, RPT)])
    @pl.when(s == NS - 1)
    def _():
        pltpu.sync_copy(s_sp.at[pl.ds(NS * RPT, TAIL)],
                        s_out.at[c, pl.ds(NS * RPT, TAIL)])


_BLK = 1000
_GRID = N // _BLK

_row_spec = pl.BlockSpec((_BLK, D), lambda i: (i, 0))
_deg_spec = pl.BlockSpec((_BLK, D), lambda i: (i, 0))
_w_spec = pl.BlockSpec((D, D), lambda i: (0, 0))
_b_spec = pl.BlockSpec((1, D), lambda i: (0, 0))


def _dinv(dega_ref, degb_ref):
    d = dega_ref[:, 0:1] + degb_ref[:, 0:1] + 1.0
    return lax.rsqrt(d)


def _tc_prep_body(dega_ref, degb_ref, x_ref, w_ref, g_ref):
    dinv = _dinv(dega_ref, degb_ref)
    h = jnp.dot(x_ref[...], w_ref[...], preferred_element_type=jnp.float32)
    g_ref[...] = dinv * h


_tc_prep = pl.pallas_call(
    _tc_prep_body,
    grid=(_GRID,),
    in_specs=[_deg_spec, _deg_spec, _row_spec, _w_spec],
    out_specs=_row_spec,
    out_shape=jax.ShapeDtypeStruct((N, D), jnp.float32),
)


def _tc_mid_body(sa_ref, sb_ref, g1_ref, dega_ref, degb_ref, b_ref, w_ref, g2_ref):
    dinv = _dinv(dega_ref, degb_ref)
    s = sa_ref[...] + sb_ref[...] + g1_ref[...]
    z = jnp.maximum(dinv * s + b_ref[...], 0.0)
    h = jnp.dot(z, w_ref[...], preferred_element_type=jnp.float32)
    g2_ref[...] = dinv * h


_tc_mid = pl.pallas_call(
    _tc_mid_body,
    grid=(_GRID,),
    in_specs=[_row_spec, _row_spec, _row_spec, _deg_spec, _deg_spec, _b_spec,
              _w_spec],
    out_specs=_row_spec,
    out_shape=jax.ShapeDtypeStruct((N, D), jnp.float32),
)


def _tc_final_body(sa_ref, sb_ref, g2_ref, dega_ref, degb_ref, b_ref, o_ref):
    dinv = _dinv(dega_ref, degb_ref)
    s = sa_ref[...] + sb_ref[...] + g2_ref[...]
    o_ref[...] = dinv * s + b_ref[...]


_tc_final = pl.pallas_call(
    _tc_final_body,
    grid=(_GRID,),
    in_specs=[_row_spec, _row_spec, _row_spec, _deg_spec, _deg_spec, _b_spec],
    out_specs=_row_spec,
    out_shape=jax.ShapeDtypeStruct((N, D), jnp.float32),
)


def kernel(x, edge_index, W1, b1, W2, b2):
    rows = edge_index[0]
    cols = edge_index[1]
    zerosd = jnp.zeros((N, D), jnp.float32)
    onesd = jnp.ones((CH, D), jnp.float32)
    b1r = b1.reshape(1, D)
    b2r = b2.reshape(1, D)

    deg_p = _sc_degree(cols, zerosd, onesd)
    dega, degb = deg_p[0], deg_p[1]

    g1 = _tc_prep(dega, degb, x, W1)
    s1 = _sc_scatter(g1, rows, cols, zerosd)
    g2 = _tc_mid(s1[0], s1[1], g1, dega, degb, b1r, W2)
    s2 = _sc_scatter(g2, rows, cols, zerosd)
    return _tc_final(s2[0], s2[1], g2, dega, degb, b2r)
